# Initial kernel scaffold; baseline (speedup 1.0000x reference)
import jax
import jax.numpy as jnp
from jax.experimental import pallas as pl


def kernel(x, emb, W_ih0, W_hh0, b_ih0, b_hh0, W_ih1, W_hh1, b_ih1, b_hh1, W_ih2, W_hh2, b_ih2, b_hh2, ln_g, ln_b, lin_W, lin_b):
    raise NotImplementedError("write your pallas kernel here")


# speedup vs baseline: 1.4536x; 1.4536x over previous
import functools

import jax
import jax.numpy as jnp
from jax import lax
from jax.experimental import pallas as pl
from jax.experimental.pallas import tpu as pltpu
from jax.experimental.pallas import tpu_sc as plsc

V = 100000
E = 128
H = 256
B = 1024
T = 50
G4 = 4 * H

NC, NS = 2, 16
NW = NC * NS
BT = B * T
B_PER_W = BT // NW
CHUNK = 400
N_CHUNK = B_PER_W // CHUNK


def _sc_gather(emb, idx_flat):
    mesh = plsc.VectorSubcoreMesh(core_axis_name="c", subcore_axis_name="s")

    @functools.partial(
        pl.kernel,
        mesh=mesh,
        out_type=jax.ShapeDtypeStruct((BT, E), jnp.float32),
        scratch_types=[
            pltpu.VMEM((CHUNK,), jnp.int32),
            pltpu.VMEM((CHUNK, E), jnp.float32),
            pltpu.SemaphoreType.DMA,
        ],
    )
    def gather_k(table_hbm, idx_hbm, out_hbm, idx_v, rows_v, sem):
        wid = lax.axis_index("s") * NC + lax.axis_index("c")
        base = wid * B_PER_W

        def body(j, carry):
            off = base + j * CHUNK
            pltpu.sync_copy(idx_hbm.at[pl.ds(off, CHUNK)], idx_v)
            pltpu.async_copy(table_hbm.at[idx_v], rows_v, sem).wait()
            pltpu.sync_copy(rows_v, out_hbm.at[pl.ds(off, CHUNK)])
            return carry

        lax.fori_loop(0, N_CHUNK, body, 0)

    return gather_k(emb, idx_flat)


def _lstm_body(x_ref, wih_ref, whh_ref, bih_ref, bhh_ref, h_sc, c_sc):
    t = pl.program_id(0)

    @pl.when(t == 0)
    def _init():
        h_sc[...] = jnp.zeros((B, H), jnp.float32)
        c_sc[...] = jnp.zeros((B, H), jnp.float32)

    x = x_ref[0]
    h = h_sc[...]
    gates = (
        lax.dot_general(x, wih_ref[...], (((1,), (1,)), ((), ())),
                        preferred_element_type=jnp.float32)
        + lax.dot_general(h, whh_ref[...], (((1,), (1,)), ((), ())),
                          preferred_element_type=jnp.float32)
        + bih_ref[...] + bhh_ref[...]
    )
    i = jax.nn.sigmoid(gates[:, 0:H])
    f = jax.nn.sigmoid(gates[:, H:2 * H])
    g = jnp.tanh(gates[:, 2 * H:3 * H])
    o = jax.nn.sigmoid(gates[:, 3 * H:4 * H])
    c = f * c_sc[...] + i * g
    hnew = o * jnp.tanh(c)
    h_sc[...] = hnew
    c_sc[...] = c
    return hnew, c


def _lstm_layer(xs, Wih, Whh, bih, bhh, interpret=False):
    Din = xs.shape[-1]

    def body(x_ref, wih_ref, whh_ref, bih_ref, bhh_ref,
             ys_ref, ht_ref, ct_ref, h_sc, c_sc):
        hnew, c = _lstm_body(x_ref, wih_ref, whh_ref, bih_ref, bhh_ref,
                             h_sc, c_sc)
        ys_ref[0] = hnew

        @pl.when(pl.program_id(0) == T - 1)
        def _fin():
            ht_ref[...] = hnew
            ct_ref[...] = c

    return pl.pallas_call(
        body,
        grid=(T,),
        in_specs=[
            pl.BlockSpec((1, B, Din), lambda t: (t, 0, 0)),
            pl.BlockSpec((G4, Din), lambda t: (0, 0)),
            pl.BlockSpec((G4, H), lambda t: (0, 0)),
            pl.BlockSpec((1, G4), lambda t: (0, 0)),
            pl.BlockSpec((1, G4), lambda t: (0, 0)),
        ],
        out_specs=[
            pl.BlockSpec((1, B, H), lambda t: (t, 0, 0)),
            pl.BlockSpec((B, H), lambda t: (0, 0)),
            pl.BlockSpec((B, H), lambda t: (0, 0)),
        ],
        out_shape=[
            jax.ShapeDtypeStruct((T, B, H), jnp.float32),
            jax.ShapeDtypeStruct((B, H), jnp.float32),
            jax.ShapeDtypeStruct((B, H), jnp.float32),
        ],
        scratch_shapes=[
            pltpu.VMEM((B, H), jnp.float32),
            pltpu.VMEM((B, H), jnp.float32),
        ],
        interpret=interpret,
    )(xs, Wih, Whh, bih.reshape(1, G4), bhh.reshape(1, G4))


def _lstm_last_layer(xs, Wih, Whh, bih, bhh, ln_g, ln_b, interpret=False):
    def body(x_ref, wih_ref, whh_ref, bih_ref, bhh_ref, lng_ref, lnb_ref,
             ht_ref, ct_ref, nrm_ref, h_sc, c_sc):
        hnew, c = _lstm_body(x_ref, wih_ref, whh_ref, bih_ref, bhh_ref,
                             h_sc, c_sc)

        @pl.when(pl.program_id(0) == T - 1)
        def _fin():
            ht_ref[...] = hnew
            ct_ref[...] = c
            mu = jnp.mean(hnew, axis=-1, keepdims=True)
            var = jnp.mean((hnew - mu) ** 2, axis=-1, keepdims=True)
            nrm_ref[...] = ((hnew - mu) * lax.rsqrt(var + 1e-5)
                            * lng_ref[...] + lnb_ref[...])

    return pl.pallas_call(
        body,
        grid=(T,),
        in_specs=[
            pl.BlockSpec((1, B, H), lambda t: (t, 0, 0)),
            pl.BlockSpec((G4, H), lambda t: (0, 0)),
            pl.BlockSpec((G4, H), lambda t: (0, 0)),
            pl.BlockSpec((1, G4), lambda t: (0, 0)),
            pl.BlockSpec((1, G4), lambda t: (0, 0)),
            pl.BlockSpec((1, H), lambda t: (0, 0)),
            pl.BlockSpec((1, H), lambda t: (0, 0)),
        ],
        out_specs=[
            pl.BlockSpec((B, H), lambda t: (0, 0)),
            pl.BlockSpec((B, H), lambda t: (0, 0)),
            pl.BlockSpec((B, H), lambda t: (0, 0)),
        ],
        out_shape=[
            jax.ShapeDtypeStruct((B, H), jnp.float32),
            jax.ShapeDtypeStruct((B, H), jnp.float32),
            jax.ShapeDtypeStruct((B, H), jnp.float32),
        ],
        scratch_shapes=[
            pltpu.VMEM((B, H), jnp.float32),
            pltpu.VMEM((B, H), jnp.float32),
        ],
        interpret=interpret,
    )(xs, Wih, Whh, bih.reshape(1, G4), bhh.reshape(1, G4),
      ln_g.reshape(1, H), ln_b.reshape(1, H))


VB = 1024
NVB = -(-V // VB)
VPAD = NVB * VB


def _head(normed, lin_W, lin_b_pad, interpret=False):
    def body(n_ref, w_ref, b_ref, out_ref):
        acc = lax.dot_general(n_ref[...], w_ref[...],
                              (((1,), (1,)), ((), ())),
                              preferred_element_type=jnp.float32)
        out_ref[...] = acc + b_ref[0]

    return pl.pallas_call(
        body,
        grid=(NVB,),
        in_specs=[
            pl.BlockSpec((B, H), lambda i: (0, 0)),
            pl.BlockSpec((VB, H), lambda i: (i, 0)),
            pl.BlockSpec((1, 1, VB), lambda i: (i, 0, 0)),
        ],
        out_specs=pl.BlockSpec((B, VB), lambda i: (0, i)),
        out_shape=jax.ShapeDtypeStruct((B, V), jnp.float32),
        interpret=interpret,
    )(normed, lin_W, lin_b_pad)


def kernel(x, emb, W_ih0, W_hh0, b_ih0, b_hh0, W_ih1, W_hh1, b_ih1, b_hh1,
           W_ih2, W_hh2, b_ih2, b_hh2, ln_g, ln_b, lin_W, lin_b):
    idx_flat = x.T.reshape(BT)
    e = _sc_gather(emb, idx_flat).reshape(T, B, E)

    ys0, h0, c0 = _lstm_layer(e, W_ih0, W_hh0, b_ih0, b_hh0)
    ys1, h1, c1 = _lstm_layer(ys0, W_ih1, W_hh1, b_ih1, b_hh1)
    h2, c2, normed = _lstm_last_layer(ys1, W_ih2, W_hh2, b_ih2, b_hh2,
                                      ln_g, ln_b)

    lin_b_pad = jnp.zeros((VPAD,), jnp.float32).at[:V].set(lin_b)
    logits = _head(normed, lin_W, lin_b_pad.reshape(NVB, 1, VB))

    hidden = (jnp.stack([h0, h1, h2]), jnp.stack([c0, c1, c2]))
    return logits, hidden
